# baseline (speedup 1.0000x reference)
import functools

import jax
import jax.numpy as jnp
import numpy as np
from jax import lax
from jax.experimental import pallas as pl
from jax.experimental.pallas import tpu as pltpu
from jax.experimental.pallas import tpu_sc as plsc

N = 10000
E = 160000
F = 78
H = 10
HF = 780
G = 512
C = 80
INC = 300

D = 80
K = 64
NBT = 88
NSC = 2
NTILE = 16
NBLK = NSC * NTILE * NBT
EPAD = NBLK * K
NPAD = 10240
RPT = NPAD // NTILE
ZR = 32


def _spmm_body(nchunks, tab_ref, w_ref, srcb_ref, dstb_ref, out_ref,
               src_t, dst_t, w_t, rows_v, zbuf, acc):
    c = lax.axis_index("c")
    s = lax.axis_index("s")
    base_blk = c * (NBLK // 2) + s * NBT
    r0 = s * RPT
    pltpu.sync_copy(srcb_ref.at[pl.ds(base_blk, NBT)], src_t)
    pltpu.sync_copy(dstb_ref.at[pl.ds(base_blk, NBT)], dst_t)
    for r in range(ZR):
        for j in range(D // 16):
            zbuf[r, pl.ds(j * 16, 16)] = jnp.zeros((16,), jnp.float32)

    def chunk_body(ci, carry):
        def zloop(z, cz):
            pltpu.sync_copy(zbuf, acc.at[pl.ds(r0 + z * ZR, ZR)])
            return cz
        lax.fori_loop(0, RPT // ZR, zloop, 0)
        plsc.subcore_barrier()
        pltpu.sync_copy(w_ref.at[ci, pl.ds(base_blk, NBT)], w_t)

        def blk_body(b, cb):
            pltpu.sync_copy(tab_ref.at[ci].at[src_t.at[b]], rows_v)
            for g in range(K // 16):
                wblk = w_t[b, pl.ds(g * 16, 16)]
                for kk in range(16):
                    k = g * 16 + kk
                    wv = jnp.full((16,), wblk[kk], jnp.float32)
                    for j in range(D // 16):
                        sl = pl.ds(j * 16, 16)
                        rows_v[k, sl] = rows_v[k, sl] * wv
            pltpu.sync_copy(rows_v, acc.at[dst_t.at[b]], add=True)
            return cb
        lax.fori_loop(0, NBT, blk_body, 0)
        plsc.subcore_barrier()
        pltpu.sync_copy(acc.at[pl.ds(r0, RPT)],
                        out_ref.at[c * nchunks + ci, pl.ds(r0, RPT)])
        plsc.subcore_barrier()
        return carry
    lax.fori_loop(0, nchunks, chunk_body, 0)


def _make_spmm(nchunks):
    mesh = plsc.VectorSubcoreMesh(core_axis_name="c", subcore_axis_name="s")
    return pl.kernel(
        functools.partial(_spmm_body, nchunks),
        out_type=jax.ShapeDtypeStruct((NSC * nchunks, NPAD, D), jnp.float32),
        mesh=mesh,
        compiler_params=pltpu.CompilerParams(use_tc_tiling_on_sc=False),
        scratch_types=[
            pltpu.VMEM((NBT, K), jnp.int32),
            pltpu.VMEM((NBT, K), jnp.int32),
            pltpu.VMEM((NBT, K), jnp.float32),
            pltpu.VMEM((K, D), jnp.float32),
            pltpu.VMEM((ZR, D), jnp.float32),
            pltpu.VMEM_SHARED((NPAD, D), jnp.float32),
        ],
    )


def _dense_tail_body(p_ref, w1_ref, b1_ref, w2_ref, b2_ref, yt_ref, o_ref):
    t = jnp.dot(p_ref[...], w1_ref[...], preferred_element_type=jnp.float32)
    t = jnp.maximum(t + b1_ref[...][None, :], 0.0)
    t = jnp.dot(t, w2_ref[...], preferred_element_type=jnp.float32)
    t = t + b2_ref[...][None, :]
    o_ref[...] = jnp.dot(t, yt_ref[...], preferred_element_type=jnp.float32)


def _dense_tail(p, w1, b1, w2, b2, y):
    return pl.pallas_call(
        _dense_tail_body,
        out_shape=jax.ShapeDtypeStruct((G, C), jnp.float32),
    )(p, w1, b1, w2, b2, y.T)


def _gen_adj(A):
    Dd = jnp.power(A.sum(1), -0.5)
    Dm = jnp.diag(Dd)
    return jnp.matmul(jnp.matmul(A, Dm).T, Dm)


def _chunk_tables(s):
    st = s.reshape(N, H, F).transpose(1, 0, 2)
    return jnp.pad(st, ((0, 0), (0, NPAD - N), (0, D - F)))


def _unchunk(o):
    return o[:, :N, :F].transpose(1, 0, 2).reshape(N, HF)


def kernel(x, edge_index, batch, inp, gat_lin, att_src, att_dst, gat_bias,
           gcn_w, gcn_b, fc1_w, fc1_b, fc2_w, fc2_b, gc1_w, gc2_w, A):
    n = x.shape[0]
    loop = jnp.arange(n, dtype=edge_index.dtype)
    src = jnp.concatenate([edge_index[0], loop])
    dst = jnp.concatenate([edge_index[1], loop])
    etot = src.shape[0]
    srcp = jnp.pad(src, (0, EPAD - etot)).reshape(NBLK, K)
    dstp = jnp.pad(dst, (0, EPAD - etot)).reshape(NBLK, K)
    h = jnp.matmul(x, gat_lin).reshape(n, H, F)
    a_src = (h * att_src[None, :, :]).sum(-1)
    a_dst = (h * att_dst[None, :, :]).sum(-1)
    e = jax.nn.leaky_relu(a_src[src] + a_dst[dst], negative_slope=0.2)
    m = jax.ops.segment_max(e, dst, num_segments=n)
    e = jnp.exp(e - m[dst])
    denom = jax.ops.segment_sum(e, dst, num_segments=n)
    alpha = e / (denom[dst] + 1e-16)
    msg = h[src] * alpha[:, :, None]
    x1 = jax.ops.segment_sum(msg, dst, num_segments=n).reshape(n, HF) + gat_bias
    x1 = jax.nn.relu(x1)
    ones = jnp.ones(src.shape, dtype=jnp.float32)
    deg = jax.ops.segment_sum(ones, dst, num_segments=n)
    dinv = jnp.where(deg > 0, jnp.power(deg, -0.5), 0.0)
    norm = dinv[src] * dinv[dst]
    s = jnp.matmul(x1, gcn_w)
    stab = _chunk_tables(s)
    wp = jnp.pad(norm, (0, EPAD - etot))
    w10 = jnp.broadcast_to(wp[None, :], (H, EPAD)).reshape(H, NBLK, K)
    out = _make_spmm(H)(stab, w10, srcp, dstp)
    x2 = _unchunk(out[:H] + out[H:]) + gcn_b
    x2 = jax.nn.relu(x2)
    gm = jax.ops.segment_max(x2, batch, num_segments=G)
    gm = jnp.where(jnp.isfinite(gm), gm, 0.0)
    cnt = jax.ops.segment_sum(jnp.ones((n,), dtype=jnp.float32), batch, num_segments=G)
    ga = jax.ops.segment_sum(x2, batch, num_segments=G) / jnp.maximum(cnt, 1.0)[:, None]
    p = jnp.concatenate([gm, ga], axis=1)
    adj = jax.lax.stop_gradient(_gen_adj(A))
    y = jnp.matmul(adj, jnp.matmul(inp, gc1_w))
    y = jax.nn.leaky_relu(y, negative_slope=0.2)
    y = jnp.matmul(adj, jnp.matmul(y, gc2_w))
    return _dense_tail(p, fc1_w, fc1_b, fc2_w, fc2_b, y)


# speedup vs baseline: 2.0563x; 2.0563x over previous
import functools

import jax
import jax.numpy as jnp
import numpy as np
from jax import lax
from jax.experimental import pallas as pl
from jax.experimental.pallas import tpu as pltpu
from jax.experimental.pallas import tpu_sc as plsc

N = 10000
E = 160000
F = 78
H = 10
HF = 780
G = 512
C = 80
INC = 300

D = 80
K = 64
NBT = 88
NSC = 2
NTILE = 16
NBLK = NSC * NTILE * NBT
EPAD = NBLK * K
NPAD = 10240
RPT = NPAD // NTILE
ZR = 32


def _spmm_body(nchunks, tab_ref, w_ref, srcb_ref, dstb_ref, out_ref,
               src_t, dst_t, w_t, rows_v, zbuf, acc):
    c = lax.axis_index("c")
    s = lax.axis_index("s")
    base_blk = c * (NBLK // 2) + s * NBT
    r0 = s * RPT
    pltpu.sync_copy(srcb_ref.at[pl.ds(base_blk, NBT)], src_t)
    pltpu.sync_copy(dstb_ref.at[pl.ds(base_blk, NBT)], dst_t)
    for r in range(ZR):
        for j in range(D // 16):
            zbuf[r, pl.ds(j * 16, 16)] = jnp.zeros((16,), jnp.float32)

    def chunk_body(ci, carry):
        def zloop(z, cz):
            pltpu.sync_copy(zbuf, acc.at[pl.ds(r0 + z * ZR, ZR)])
            return cz
        lax.fori_loop(0, RPT // ZR, zloop, 0)
        plsc.subcore_barrier()
        pltpu.sync_copy(w_ref.at[ci, pl.ds(base_blk, NBT)], w_t)

        def blk_body(b, cb):
            pltpu.sync_copy(tab_ref.at[ci].at[src_t.at[b]], rows_v)
            for g in range(K // 16):
                wblk = w_t[b, pl.ds(g * 16, 16)]
                for kk in range(16):
                    k = g * 16 + kk
                    wv = jnp.full((16,), wblk[kk], jnp.float32)
                    for j in range(D // 16):
                        sl = pl.ds(j * 16, 16)
                        rows_v[k, sl] = rows_v[k, sl] * wv
            pltpu.sync_copy(rows_v, acc.at[dst_t.at[b]], add=True)
            return cb
        lax.fori_loop(0, NBT, blk_body, 0)
        plsc.subcore_barrier()
        pltpu.sync_copy(acc.at[pl.ds(r0, RPT)],
                        out_ref.at[c * nchunks + ci, pl.ds(r0, RPT)])
        plsc.subcore_barrier()
        return carry
    lax.fori_loop(0, nchunks, chunk_body, 0)


def _make_spmm(nchunks):
    mesh = plsc.VectorSubcoreMesh(core_axis_name="c", subcore_axis_name="s")
    return pl.kernel(
        functools.partial(_spmm_body, nchunks),
        out_type=jax.ShapeDtypeStruct((NSC * nchunks, NPAD, D), jnp.float32),
        mesh=mesh,
        compiler_params=pltpu.CompilerParams(use_tc_tiling_on_sc=False),
        scratch_types=[
            pltpu.VMEM((NBT, K), jnp.int32),
            pltpu.VMEM((NBT, K), jnp.int32),
            pltpu.VMEM((NBT, K), jnp.float32),
            pltpu.VMEM((K, D), jnp.float32),
            pltpu.VMEM((ZR, D), jnp.float32),
            pltpu.VMEM_SHARED((NPAD, D), jnp.float32),
        ],
    )


def _dense_tail_body(p_ref, w1_ref, b1_ref, w2_ref, b2_ref, yt_ref, o_ref):
    t = jnp.dot(p_ref[...], w1_ref[...], preferred_element_type=jnp.float32)
    t = jnp.maximum(t + b1_ref[...][None, :], 0.0)
    t = jnp.dot(t, w2_ref[...], preferred_element_type=jnp.float32)
    t = t + b2_ref[...][None, :]
    o_ref[...] = jnp.dot(t, yt_ref[...], preferred_element_type=jnp.float32)


def _dense_tail(p, w1, b1, w2, b2, y):
    return pl.pallas_call(
        _dense_tail_body,
        out_shape=jax.ShapeDtypeStruct((G, C), jnp.float32),
    )(p, w1, b1, w2, b2, y.T)


def _gen_adj(A):
    Dd = jnp.power(A.sum(1), -0.5)
    Dm = jnp.diag(Dd)
    return jnp.matmul(jnp.matmul(A, Dm).T, Dm)


def _chunk_tables(s):
    st = s.reshape(N, H, F).transpose(1, 0, 2)
    return jnp.pad(st, ((0, 0), (0, NPAD - N), (0, D - F)))


def _unchunk(o):
    return o[:, :N, :F].transpose(1, 0, 2).reshape(N, HF)


def kernel(x, edge_index, batch, inp, gat_lin, att_src, att_dst, gat_bias,
           gcn_w, gcn_b, fc1_w, fc1_b, fc2_w, fc2_b, gc1_w, gc2_w, A):
    n = x.shape[0]
    loop = jnp.arange(n, dtype=edge_index.dtype)
    src = jnp.concatenate([edge_index[0], loop])
    dst = jnp.concatenate([edge_index[1], loop])
    etot = src.shape[0]
    srcp = jnp.pad(src, (0, EPAD - etot)).reshape(NBLK, K)
    dstp = jnp.pad(dst, (0, EPAD - etot)).reshape(NBLK, K)
    h = jnp.matmul(x, gat_lin).reshape(n, H, F)
    a_src = (h * att_src[None, :, :]).sum(-1)
    a_dst = (h * att_dst[None, :, :]).sum(-1)
    e = jax.nn.leaky_relu(a_src[src] + a_dst[dst], negative_slope=0.2)
    m = jax.ops.segment_max(e, dst, num_segments=n)
    e = jnp.exp(e - m[dst])
    denom = jax.ops.segment_sum(e, dst, num_segments=n)
    alpha = e / (denom[dst] + 1e-16)
    htab = _chunk_tables(h.reshape(n, HF))
    walpha = jnp.pad(alpha.T, ((0, 0), (0, EPAD - etot))).reshape(H, NBLK, K)
    outg = _make_spmm(H)(htab, walpha, srcp, dstp)
    x1 = _unchunk(outg[:H] + outg[H:]) + gat_bias
    x1 = jax.nn.relu(x1)
    ones = jnp.ones(src.shape, dtype=jnp.float32)
    deg = jax.ops.segment_sum(ones, dst, num_segments=n)
    dinv = jnp.where(deg > 0, jnp.power(deg, -0.5), 0.0)
    norm = dinv[src] * dinv[dst]
    s = jnp.matmul(x1, gcn_w)
    stab = _chunk_tables(s)
    wp = jnp.pad(norm, (0, EPAD - etot))
    w10 = jnp.broadcast_to(wp[None, :], (H, EPAD)).reshape(H, NBLK, K)
    out = _make_spmm(H)(stab, w10, srcp, dstp)
    x2 = _unchunk(out[:H] + out[H:]) + gcn_b
    x2 = jax.nn.relu(x2)
    gm = jax.ops.segment_max(x2, batch, num_segments=G)
    gm = jnp.where(jnp.isfinite(gm), gm, 0.0)
    cnt = jax.ops.segment_sum(jnp.ones((n,), dtype=jnp.float32), batch, num_segments=G)
    ga = jax.ops.segment_sum(x2, batch, num_segments=G) / jnp.maximum(cnt, 1.0)[:, None]
    p = jnp.concatenate([gm, ga], axis=1)
    adj = jax.lax.stop_gradient(_gen_adj(A))
    y = jnp.matmul(adj, jnp.matmul(inp, gc1_w))
    y = jax.nn.leaky_relu(y, negative_slope=0.2)
    y = jnp.matmul(adj, jnp.matmul(y, gc2_w))
    return _dense_tail(p, fc1_w, fc1_b, fc2_w, fc2_b, y)


# speedup vs baseline: 3.4492x; 1.6774x over previous
import jax
import jax.numpy as jnp
import numpy as np
from jax import lax
from jax.experimental import pallas as pl
from jax.experimental.pallas import tpu as pltpu
from jax.experimental.pallas import tpu_sc as plsc

N = 10000
E = 160000
F = 78
H = 10
HF = 780
G = 512
C = 80
INC = 300

D = 80
K = 64
NBT = 88
NSC = 2
NTILE = 16
NBLK = NSC * NTILE * NBT
EPAD = NBLK * K
ETOT = N + E
NPAD = 10240
RPT = NPAD // NTILE
ZR = 32


def _zero_acc(zbuf, acc, r0):
    def zloop(z, cz):
        pltpu.sync_copy(zbuf, acc.at[pl.ds(r0 + z * ZR, ZR)])
        return cz
    lax.fori_loop(0, RPT // ZR, zloop, 0)


def _leaky_exp(v):
    v = jnp.where(v > 0, v, v * jnp.float32(0.2))
    return jnp.exp(v)


def _pipeline_blocks(tab_view, aview, bview, combine, src_t, dst_t,
                     rows, arows, brows, acc, semh, sema):
    def issue(b, p):
        pltpu.async_copy(tab_view.at[src_t.at[b]], rows[p], semh[p])
        pltpu.async_copy(aview.at[src_t.at[b]], arows[p], sema[p])
        pltpu.async_copy(bview.at[dst_t.at[b]], brows[p], sema[p])

    def wait(b, p):
        pltpu.make_async_copy(tab_view.at[src_t.at[b]], rows[p], semh[p]).wait()
        pltpu.make_async_copy(aview.at[src_t.at[b]], arows[p], sema[p]).wait()
        pltpu.make_async_copy(bview.at[dst_t.at[b]], brows[p], sema[p]).wait()

    def work(b, p):
        for k in range(K):
            w = combine(arows[p][k, :], brows[p][k, :])
            for j in range(D // 16):
                sl = pl.ds(j * 16, 16)
                rows[p][k, sl] = rows[p][k, sl] * w
        pltpu.sync_copy(rows[p], acc.at[dst_t.at[b]], add=True)

    issue(0, 0)

    def pair(i, cp):
        b0 = 2 * i
        issue(b0 + 1, 1)
        wait(b0, 0)
        work(b0, 0)

        @pl.when(b0 + 2 < NBT)
        def _():
            issue(b0 + 2, 0)
        wait(b0 + 1, 1)
        work(b0 + 1, 1)
        return cp
    lax.fori_loop(0, NBT // 2, pair, 0)


def _gat_body(tab_ref, asrc_ref, adst_ref, srcb_ref, dstb_ref,
              out_ref, outdeg_ref,
              src_t, dst_t, rows0, rows1, ar0, ar1, br0, br1,
              zbuf, zbuf16, ones_v, acc, accdeg, sem0, sem1, sa0, sa1):
    c = lax.axis_index("c")
    s = lax.axis_index("s")
    base_blk = c * (NBLK // 2) + s * NBT
    r0 = s * RPT
    pltpu.sync_copy(srcb_ref.at[pl.ds(base_blk, NBT)], src_t)
    pltpu.sync_copy(dstb_ref.at[pl.ds(base_blk, NBT)], dst_t)
    for r in range(ZR):
        for j in range(D // 16):
            zbuf[r, pl.ds(j * 16, 16)] = jnp.zeros((16,), jnp.float32)
        zbuf16[r, :] = jnp.zeros((16,), jnp.float32)
    for k in range(K):
        ones_v[k, :] = jnp.full((16,), 1.0, jnp.float32)

    def chunk_body(ci, carry):
        _zero_acc(zbuf, acc, r0)
        plsc.subcore_barrier()
        _pipeline_blocks(
            tab_ref.at[ci], asrc_ref.at[ci], adst_ref.at[ci],
            lambda a, b: _leaky_exp(a + b),
            src_t, dst_t, (rows0, rows1), (ar0, ar1), (br0, br1),
            acc, (sem0, sem1), (sa0, sa1))
        plsc.subcore_barrier()
        pltpu.sync_copy(acc.at[pl.ds(r0, RPT)],
                        out_ref.at[c * H + ci, pl.ds(r0, RPT)])
        plsc.subcore_barrier()
        return carry
    lax.fori_loop(0, H, chunk_body, 0)

    def zloop16(z, cz):
        pltpu.sync_copy(zbuf16, accdeg.at[pl.ds(r0 + z * ZR, ZR)])
        return cz
    lax.fori_loop(0, RPT // ZR, zloop16, 0)
    plsc.subcore_barrier()

    def deg_body(b, cd):
        pltpu.sync_copy(ones_v, accdeg.at[dst_t.at[b]], add=True)
        return cd
    lax.fori_loop(0, NBT, deg_body, 0)
    plsc.subcore_barrier()
    pltpu.sync_copy(accdeg.at[pl.ds(r0, RPT)],
                    outdeg_ref.at[c, pl.ds(r0, RPT)])
    plsc.subcore_barrier()


def _gcn_body(tab_ref, dinv_ref, srcb_ref, dstb_ref, out_ref,
              src_t, dst_t, rows0, rows1, ar0, ar1, br0, br1,
              zbuf, acc, sem0, sem1, sa0, sa1):
    c = lax.axis_index("c")
    s = lax.axis_index("s")
    base_blk = c * (NBLK // 2) + s * NBT
    r0 = s * RPT
    pltpu.sync_copy(srcb_ref.at[pl.ds(base_blk, NBT)], src_t)
    pltpu.sync_copy(dstb_ref.at[pl.ds(base_blk, NBT)], dst_t)
    for r in range(ZR):
        for j in range(D // 16):
            zbuf[r, pl.ds(j * 16, 16)] = jnp.zeros((16,), jnp.float32)

    def chunk_body(ci, carry):
        _zero_acc(zbuf, acc, r0)
        plsc.subcore_barrier()
        _pipeline_blocks(
            tab_ref.at[ci], dinv_ref, dinv_ref,
            lambda a, b: a * b,
            src_t, dst_t, (rows0, rows1), (ar0, ar1), (br0, br1),
            acc, (sem0, sem1), (sa0, sa1))
        plsc.subcore_barrier()
        pltpu.sync_copy(acc.at[pl.ds(r0, RPT)],
                        out_ref.at[c * H + ci, pl.ds(r0, RPT)])
        plsc.subcore_barrier()
        return carry
    lax.fori_loop(0, H, chunk_body, 0)


_SC_MESH = plsc.VectorSubcoreMesh(core_axis_name="c", subcore_axis_name="s")
_NO_TILING = pltpu.CompilerParams(use_tc_tiling_on_sc=False)

_SPMM_SCRATCH = [
    pltpu.VMEM((NBT, K), jnp.int32),
    pltpu.VMEM((NBT, K), jnp.int32),
    pltpu.VMEM((K, D), jnp.float32),
    pltpu.VMEM((K, D), jnp.float32),
    pltpu.VMEM((K, 16), jnp.float32),
    pltpu.VMEM((K, 16), jnp.float32),
    pltpu.VMEM((K, 16), jnp.float32),
    pltpu.VMEM((K, 16), jnp.float32),
]

_gat_spmm = pl.kernel(
    _gat_body,
    out_type=[
        jax.ShapeDtypeStruct((NSC * H, NPAD, D), jnp.float32),
        jax.ShapeDtypeStruct((NSC, NPAD, 16), jnp.float32),
    ],
    mesh=_SC_MESH,
    compiler_params=_NO_TILING,
    scratch_types=_SPMM_SCRATCH + [
        pltpu.VMEM((ZR, D), jnp.float32),
        pltpu.VMEM((ZR, 16), jnp.float32),
        pltpu.VMEM((K, 16), jnp.float32),
        pltpu.VMEM_SHARED((NPAD, D), jnp.float32),
        pltpu.VMEM_SHARED((NPAD, 16), jnp.float32),
        pltpu.SemaphoreType.DMA,
        pltpu.SemaphoreType.DMA,
        pltpu.SemaphoreType.DMA,
        pltpu.SemaphoreType.DMA,
    ],
)

_gcn_spmm = pl.kernel(
    _gcn_body,
    out_type=jax.ShapeDtypeStruct((NSC * H, NPAD, D), jnp.float32),
    mesh=_SC_MESH,
    compiler_params=_NO_TILING,
    scratch_types=_SPMM_SCRATCH + [
        pltpu.VMEM((ZR, D), jnp.float32),
        pltpu.VMEM_SHARED((NPAD, D), jnp.float32),
        pltpu.SemaphoreType.DMA,
        pltpu.SemaphoreType.DMA,
        pltpu.SemaphoreType.DMA,
        pltpu.SemaphoreType.DMA,
    ],
)


def _dense_tail_body(p_ref, w1_ref, b1_ref, w2_ref, b2_ref, yt_ref, o_ref):
    t = jnp.dot(p_ref[...], w1_ref[...], preferred_element_type=jnp.float32)
    t = jnp.maximum(t + b1_ref[...][None, :], 0.0)
    t = jnp.dot(t, w2_ref[...], preferred_element_type=jnp.float32)
    t = t + b2_ref[...][None, :]
    o_ref[...] = jnp.dot(t, yt_ref[...], preferred_element_type=jnp.float32)


def _dense_tail(p, w1, b1, w2, b2, y):
    return pl.pallas_call(
        _dense_tail_body,
        out_shape=jax.ShapeDtypeStruct((G, C), jnp.float32),
    )(p, w1, b1, w2, b2, y.T)


def _gen_adj(A):
    Dd = jnp.power(A.sum(1), -0.5)
    Dm = jnp.diag(Dd)
    return jnp.matmul(jnp.matmul(A, Dm).T, Dm)


def _chunk_tables(sarr, ones_col):
    st = sarr.reshape(N, H, F).transpose(1, 0, 2)
    st = jnp.pad(st, ((0, 0), (0, NPAD - N), (0, D - F)))
    if ones_col:
        ones = jnp.zeros((H, NPAD, D), jnp.float32).at[:, :N, F].set(1.0)
        st = st + ones
    return st


def _unchunk(o):
    return o[:, :N, :F].transpose(1, 0, 2).reshape(N, HF)


def _replicate16(a):
    return jnp.broadcast_to(a[..., None], a.shape + (16,))


def kernel(x, edge_index, batch, inp, gat_lin, att_src, att_dst, gat_bias,
           gcn_w, gcn_b, fc1_w, fc1_b, fc2_w, fc2_b, gc1_w, gc2_w, A):
    n = x.shape[0]
    loop = jnp.arange(n, dtype=edge_index.dtype)
    src = jnp.concatenate([edge_index[0], loop])
    dst = jnp.concatenate([edge_index[1], loop])
    etot = src.shape[0]
    sentinel = jnp.full((EPAD - etot,), N, jnp.int32)
    srcp = jnp.concatenate([src, sentinel]).reshape(NBLK, K)
    dstp = jnp.concatenate([dst, sentinel]).reshape(NBLK, K)
    h = jnp.matmul(x, gat_lin).reshape(n, H, F)
    a_src = (h * att_src[None, :, :]).sum(-1)
    a_dst = (h * att_dst[None, :, :]).sum(-1)
    neg = jnp.full((H, NPAD - n), -1e9, jnp.float32)
    asrcX = _replicate16(jnp.concatenate([a_src.T, neg], axis=1))
    adstX = _replicate16(jnp.concatenate([a_dst.T, neg], axis=1))
    htab = _chunk_tables(h.reshape(n, HF), ones_col=True)
    outg, outdeg = _gat_spmm(htab, asrcX, adstX, srcp, dstp)
    og = outg[:H] + outg[H:]
    den = og[:, :N, F].transpose(1, 0) + 1e-16
    x1 = _unchunk(og) / jnp.repeat(den, F, axis=1) + gat_bias
    x1 = jax.nn.relu(x1)
    deg = (outdeg[0] + outdeg[1])[:N, 0]
    dinv = jnp.where(deg > 0, jax.lax.rsqrt(deg), 0.0)
    dinvX = _replicate16(jnp.pad(dinv, (0, NPAD - n)))
    s = jnp.matmul(x1, gcn_w)
    stab = _chunk_tables(s, ones_col=False)
    out = _gcn_spmm(stab, dinvX, srcp, dstp)
    x2 = _unchunk(out[:H] + out[H:]) + gcn_b
    x2 = jax.nn.relu(x2)
    gm = jax.ops.segment_max(x2, batch, num_segments=G)
    gm = jnp.where(jnp.isfinite(gm), gm, 0.0)
    cnt = jax.ops.segment_sum(jnp.ones((n,), dtype=jnp.float32), batch, num_segments=G)
    ga = jax.ops.segment_sum(x2, batch, num_segments=G) / jnp.maximum(cnt, 1.0)[:, None]
    p = jnp.concatenate([gm, ga], axis=1)
    adj = jax.lax.stop_gradient(_gen_adj(A))
    y = jnp.matmul(adj, jnp.matmul(inp, gc1_w))
    y = jax.nn.leaky_relu(y, negative_slope=0.2)
    y = jnp.matmul(adj, jnp.matmul(y, gc2_w))
    return _dense_tail(p, fc1_w, fc1_b, fc2_w, fc2_b, y)


# speedup vs baseline: 9.3757x; 2.7182x over previous
import jax
import jax.numpy as jnp
import numpy as np
from jax import lax
from jax.experimental import pallas as pl
from jax.experimental.pallas import tpu as pltpu
from jax.experimental.pallas import tpu_sc as plsc

N = 10000
E = 160000
F = 78
H = 10
HF = 780
G = 512
C = 80
INC = 300

D = 80
K = 64
NBT = 88
NSC = 2
NTILE = 16
NBLK = NSC * NTILE * NBT
EPAD = NBLK * K
ETOT = N + E
NPAD = 10240
RPT = NPAD // NTILE
ZR = 32


def _zero_acc(zbuf, acc, r0):
    def zloop(z, cz):
        pltpu.sync_copy(zbuf, acc.at[pl.ds(r0 + z * ZR, ZR)])
        return cz
    lax.fori_loop(0, RPT // ZR, zloop, 0)


def _leaky_exp(v):
    v = jnp.where(v > 0, v, v * jnp.float32(0.2))
    return jnp.exp(v)


def _pipeline_blocks(tab_view, aview, bview, combine, src_t, dst_t,
                     rows, arows, brows, acc, semh, sema):
    def issue(b, p):
        pltpu.async_copy(tab_view.at[src_t.at[b]], rows[p], semh[p])
        pltpu.async_copy(aview.at[src_t.at[b]], arows[p], sema[p])
        pltpu.async_copy(bview.at[dst_t.at[b]], brows[p], sema[p])

    def wait(b, p):
        pltpu.make_async_copy(tab_view.at[src_t.at[b]], rows[p], semh[p]).wait()
        pltpu.make_async_copy(aview.at[src_t.at[b]], arows[p], sema[p]).wait()
        pltpu.make_async_copy(bview.at[dst_t.at[b]], brows[p], sema[p]).wait()

    def work(b, p):
        for k in range(K):
            w = combine(arows[p][k, :], brows[p][k, :])
            for j in range(D // 16):
                sl = pl.ds(j * 16, 16)
                rows[p][k, sl] = rows[p][k, sl] * w
        pltpu.sync_copy(rows[p], acc.at[dst_t.at[b]], add=True)

    issue(0, 0)

    def pair(i, cp):
        b0 = 2 * i
        issue(b0 + 1, 1)
        wait(b0, 0)
        work(b0, 0)

        @pl.when(b0 + 2 < NBT)
        def _():
            issue(b0 + 2, 0)
        wait(b0 + 1, 1)
        work(b0 + 1, 1)
        return cp
    lax.fori_loop(0, NBT // 2, pair, 0)


def _gat_body(tab_ref, asrc_ref, adst_ref, srcb_ref, dstb_ref,
              out_ref, outdeg_ref,
              src_t, dst_t, rows0, rows1, ar0, ar1, br0, br1,
              zbuf, zbuf16, ones_v, acc, accdeg, sem0, sem1, sa0, sa1):
    c = lax.axis_index("c")
    s = lax.axis_index("s")
    base_blk = c * (NBLK // 2) + s * NBT
    r0 = s * RPT
    pltpu.sync_copy(srcb_ref.at[pl.ds(base_blk, NBT)], src_t)
    pltpu.sync_copy(dstb_ref.at[pl.ds(base_blk, NBT)], dst_t)
    for r in range(ZR):
        for j in range(D // 16):
            zbuf[r, pl.ds(j * 16, 16)] = jnp.zeros((16,), jnp.float32)
        zbuf16[r, :] = jnp.zeros((16,), jnp.float32)
    for k in range(K):
        ones_v[k, :] = jnp.full((16,), 1.0, jnp.float32)

    def chunk_body(ci, carry):
        _zero_acc(zbuf, acc, r0)
        plsc.subcore_barrier()
        _pipeline_blocks(
            tab_ref.at[ci], asrc_ref.at[ci], adst_ref.at[ci],
            lambda a, b: _leaky_exp(a + b),
            src_t, dst_t, (rows0, rows1), (ar0, ar1), (br0, br1),
            acc, (sem0, sem1), (sa0, sa1))
        plsc.subcore_barrier()
        pltpu.sync_copy(acc.at[pl.ds(r0, RPT)],
                        out_ref.at[c * H + ci, pl.ds(r0, RPT)])
        plsc.subcore_barrier()
        return carry
    lax.fori_loop(0, H, chunk_body, 0)

    def zloop16(z, cz):
        pltpu.sync_copy(zbuf16, accdeg.at[pl.ds(r0 + z * ZR, ZR)])
        return cz
    lax.fori_loop(0, RPT // ZR, zloop16, 0)
    plsc.subcore_barrier()

    def deg_body(b, cd):
        pltpu.sync_copy(ones_v, accdeg.at[dst_t.at[b]], add=True)
        return cd
    lax.fori_loop(0, NBT, deg_body, 0)
    plsc.subcore_barrier()
    pltpu.sync_copy(accdeg.at[pl.ds(r0, RPT)],
                    outdeg_ref.at[c, pl.ds(r0, RPT)])
    plsc.subcore_barrier()


def _gcn_body(tab_ref, dinv_ref, srcb_ref, dstb_ref, out_ref,
              src_t, dst_t, rows0, rows1, ar0, ar1, br0, br1,
              zbuf, acc, sem0, sem1, sa0, sa1):
    c = lax.axis_index("c")
    s = lax.axis_index("s")
    base_blk = c * (NBLK // 2) + s * NBT
    r0 = s * RPT
    pltpu.sync_copy(srcb_ref.at[pl.ds(base_blk, NBT)], src_t)
    pltpu.sync_copy(dstb_ref.at[pl.ds(base_blk, NBT)], dst_t)
    for r in range(ZR):
        for j in range(D // 16):
            zbuf[r, pl.ds(j * 16, 16)] = jnp.zeros((16,), jnp.float32)

    def chunk_body(ci, carry):
        _zero_acc(zbuf, acc, r0)
        plsc.subcore_barrier()
        _pipeline_blocks(
            tab_ref.at[ci], dinv_ref, dinv_ref,
            lambda a, b: a * b,
            src_t, dst_t, (rows0, rows1), (ar0, ar1), (br0, br1),
            acc, (sem0, sem1), (sa0, sa1))
        plsc.subcore_barrier()
        pltpu.sync_copy(acc.at[pl.ds(r0, RPT)],
                        out_ref.at[c * H + ci, pl.ds(r0, RPT)])
        plsc.subcore_barrier()
        return carry
    lax.fori_loop(0, H, chunk_body, 0)


_SC_MESH = plsc.VectorSubcoreMesh(core_axis_name="c", subcore_axis_name="s")
_NO_TILING = pltpu.CompilerParams(use_tc_tiling_on_sc=False)

_SPMM_SCRATCH = [
    pltpu.VMEM((NBT, K), jnp.int32),
    pltpu.VMEM((NBT, K), jnp.int32),
    pltpu.VMEM((K, D), jnp.float32),
    pltpu.VMEM((K, D), jnp.float32),
    pltpu.VMEM((K, 16), jnp.float32),
    pltpu.VMEM((K, 16), jnp.float32),
    pltpu.VMEM((K, 16), jnp.float32),
    pltpu.VMEM((K, 16), jnp.float32),
]

_gat_spmm = pl.kernel(
    _gat_body,
    out_type=[
        jax.ShapeDtypeStruct((NSC * H, NPAD, D), jnp.float32),
        jax.ShapeDtypeStruct((NSC, NPAD, 16), jnp.float32),
    ],
    mesh=_SC_MESH,
    compiler_params=_NO_TILING,
    scratch_types=_SPMM_SCRATCH + [
        pltpu.VMEM((ZR, D), jnp.float32),
        pltpu.VMEM((ZR, 16), jnp.float32),
        pltpu.VMEM((K, 16), jnp.float32),
        pltpu.VMEM_SHARED((NPAD, D), jnp.float32),
        pltpu.VMEM_SHARED((NPAD, 16), jnp.float32),
        pltpu.SemaphoreType.DMA,
        pltpu.SemaphoreType.DMA,
        pltpu.SemaphoreType.DMA,
        pltpu.SemaphoreType.DMA,
    ],
)

_gcn_spmm = pl.kernel(
    _gcn_body,
    out_type=jax.ShapeDtypeStruct((NSC * H, NPAD, D), jnp.float32),
    mesh=_SC_MESH,
    compiler_params=_NO_TILING,
    scratch_types=_SPMM_SCRATCH + [
        pltpu.VMEM((ZR, D), jnp.float32),
        pltpu.VMEM_SHARED((NPAD, D), jnp.float32),
        pltpu.SemaphoreType.DMA,
        pltpu.SemaphoreType.DMA,
        pltpu.SemaphoreType.DMA,
        pltpu.SemaphoreType.DMA,
    ],
)


def _dense_tail_body(p_ref, w1_ref, b1_ref, w2_ref, b2_ref, yt_ref, o_ref):
    t = jnp.dot(p_ref[...], w1_ref[...], preferred_element_type=jnp.float32)
    t = jnp.maximum(t + b1_ref[...][None, :], 0.0)
    t = jnp.dot(t, w2_ref[...], preferred_element_type=jnp.float32)
    t = t + b2_ref[...][None, :]
    o_ref[...] = jnp.dot(t, yt_ref[...], preferred_element_type=jnp.float32)


def _dense_tail(p, w1, b1, w2, b2, y):
    return pl.pallas_call(
        _dense_tail_body,
        out_shape=jax.ShapeDtypeStruct((G, C), jnp.float32),
    )(p, w1, b1, w2, b2, y.T)


def _gen_adj(A):
    Dd = jnp.power(A.sum(1), -0.5)
    Dm = jnp.diag(Dd)
    return jnp.matmul(jnp.matmul(A, Dm).T, Dm)


def _chunk_tables(sarr, ones_col):
    st = sarr.reshape(N, H, F).transpose(1, 0, 2)
    st = jnp.pad(st, ((0, 0), (0, NPAD - N), (0, D - F)))
    if ones_col:
        ones = jnp.zeros((H, NPAD, D), jnp.float32).at[:, :N, F].set(1.0)
        st = st + ones
    return st


def _unchunk(o):
    return o[:, :N, :F].transpose(1, 0, 2).reshape(N, HF)


def _replicate16(a):
    return jnp.broadcast_to(a[..., None], a.shape + (16,))


def kernel(x, edge_index, batch, inp, gat_lin, att_src, att_dst, gat_bias,
           gcn_w, gcn_b, fc1_w, fc1_b, fc2_w, fc2_b, gc1_w, gc2_w, A):
    n = x.shape[0]
    loop = jnp.arange(n, dtype=edge_index.dtype)
    src = jnp.concatenate([edge_index[0], loop])
    dst = jnp.concatenate([edge_index[1], loop])
    etot = src.shape[0]
    sentinel = N + jnp.arange(EPAD - etot, dtype=jnp.int32) % (NPAD - N)
    srcp = jnp.concatenate([src, sentinel]).reshape(NBLK, K)
    dstp = jnp.concatenate([dst, sentinel]).reshape(NBLK, K)
    h = jnp.matmul(x, gat_lin).reshape(n, H, F)
    a_src = (h * att_src[None, :, :]).sum(-1)
    a_dst = (h * att_dst[None, :, :]).sum(-1)
    neg = jnp.full((H, NPAD - n), -1e9, jnp.float32)
    asrcX = _replicate16(jnp.concatenate([a_src.T, neg], axis=1))
    adstX = _replicate16(jnp.concatenate([a_dst.T, neg], axis=1))
    htab = _chunk_tables(h.reshape(n, HF), ones_col=True)
    outg, outdeg = _gat_spmm(htab, asrcX, adstX, srcp, dstp)
    og = outg[:H] + outg[H:]
    den = og[:, :N, F].transpose(1, 0) + 1e-16
    x1 = _unchunk(og) / jnp.repeat(den, F, axis=1) + gat_bias
    x1 = jax.nn.relu(x1)
    deg = (outdeg[0] + outdeg[1])[:N, 0]
    dinv = jnp.where(deg > 0, jax.lax.rsqrt(deg), 0.0)
    dinvX = _replicate16(jnp.pad(dinv, (0, NPAD - n)))
    s = jnp.matmul(x1, gcn_w)
    stab = _chunk_tables(s, ones_col=False)
    out = _gcn_spmm(stab, dinvX, srcp, dstp)
    x2 = _unchunk(out[:H] + out[H:]) + gcn_b
    x2 = jax.nn.relu(x2)
    gm = jax.ops.segment_max(x2, batch, num_segments=G)
    gm = jnp.where(jnp.isfinite(gm), gm, 0.0)
    cnt = jax.ops.segment_sum(jnp.ones((n,), dtype=jnp.float32), batch, num_segments=G)
    ga = jax.ops.segment_sum(x2, batch, num_segments=G) / jnp.maximum(cnt, 1.0)[:, None]
    p = jnp.concatenate([gm, ga], axis=1)
    adj = jax.lax.stop_gradient(_gen_adj(A))
    y = jnp.matmul(adj, jnp.matmul(inp, gc1_w))
    y = jax.nn.leaky_relu(y, negative_slope=0.2)
    y = jnp.matmul(adj, jnp.matmul(y, gc2_w))
    return _dense_tail(p, fc1_w, fc1_b, fc2_w, fc2_b, y)


# speedup vs baseline: 10.2302x; 1.0911x over previous
import jax
import jax.numpy as jnp
import numpy as np
from jax import lax
from jax.experimental import pallas as pl
from jax.experimental.pallas import tpu as pltpu
from jax.experimental.pallas import tpu_sc as plsc

N = 10000
E = 160000
F = 78
H = 10
HF = 780
G = 512
C = 80
INC = 300

D = 80
K = 64
NBT = 88
NSC = 2
NTILE = 16
NBLK = NSC * NTILE * NBT
EPAD = NBLK * K
ETOT = N + E
NPAD = 10240
RPT = NPAD // NTILE
ZR = 32


def _zero_acc(zbuf, acc, r0):
    def zloop(z, cz):
        pltpu.sync_copy(zbuf, acc.at[pl.ds(r0 + z * ZR, ZR)])
        return cz
    lax.fori_loop(0, RPT // ZR, zloop, 0)


def _leaky_exp(v):
    v = jnp.where(v > 0, v, v * jnp.float32(0.2))
    return jnp.exp(v)


def _pipeline_blocks(tab_view, aview, bview, combine, src_t, dst_t,
                     rows, arows, brows, acc, semh, sema):
    def issue(b, p):
        pltpu.async_copy(tab_view.at[src_t.at[b]], rows[p], semh[p])
        pltpu.async_copy(aview.at[src_t.at[b]], arows[p], sema[p])
        pltpu.async_copy(bview.at[dst_t.at[b]], brows[p], sema[p])

    def wait(b, p):
        pltpu.make_async_copy(tab_view.at[src_t.at[b]], rows[p], semh[p]).wait()
        pltpu.make_async_copy(aview.at[src_t.at[b]], arows[p], sema[p]).wait()
        pltpu.make_async_copy(bview.at[dst_t.at[b]], brows[p], sema[p]).wait()

    def work(b, p):
        for k in range(K):
            w = combine(arows[p][k, :], brows[p][k, :])
            for j in range(D // 16):
                sl = pl.ds(j * 16, 16)
                rows[p][k, sl] = rows[p][k, sl] * w
        pltpu.sync_copy(rows[p], acc.at[dst_t.at[b]], add=True)

    issue(0, 0)

    def pair(i, cp):
        b0 = 2 * i
        issue(b0 + 1, 1)
        wait(b0, 0)
        work(b0, 0)

        @pl.when(b0 + 2 < NBT)
        def _():
            issue(b0 + 2, 0)
        wait(b0 + 1, 1)
        work(b0 + 1, 1)
        return cp
    lax.fori_loop(0, NBT // 2, pair, 0)


def _gat_body(tab_ref, asrc_ref, adst_ref, srcb_ref, dstb_ref,
              out_ref, outdeg_ref,
              src_t, dst_t, rows0, rows1, ar0, ar1, br0, br1,
              zbuf, zbuf16, ones_v, acc, accdeg, sem0, sem1, sa0, sa1):
    c = lax.axis_index("c")
    s = lax.axis_index("s")
    base_blk = c * (NBLK // 2) + s * NBT
    r0 = s * RPT
    pltpu.sync_copy(srcb_ref.at[pl.ds(base_blk, NBT)], src_t)
    pltpu.sync_copy(dstb_ref.at[pl.ds(base_blk, NBT)], dst_t)
    for r in range(ZR):
        for j in range(D // 16):
            zbuf[r, pl.ds(j * 16, 16)] = jnp.zeros((16,), jnp.float32)
        zbuf16[r, :] = jnp.zeros((16,), jnp.float32)
    for k in range(K):
        ones_v[k, :] = jnp.full((16,), 1.0, jnp.float32)

    def chunk_body(ci, carry):
        _zero_acc(zbuf, acc, r0)
        plsc.subcore_barrier()
        _pipeline_blocks(
            tab_ref.at[ci], asrc_ref.at[ci], adst_ref.at[ci],
            lambda a, b: _leaky_exp(a + b),
            src_t, dst_t, (rows0, rows1), (ar0, ar1), (br0, br1),
            acc, (sem0, sem1), (sa0, sa1))
        plsc.subcore_barrier()
        pltpu.sync_copy(acc.at[pl.ds(r0, RPT)],
                        out_ref.at[c * H + ci, pl.ds(r0, RPT)])
        plsc.subcore_barrier()
        return carry
    lax.fori_loop(0, H, chunk_body, 0)

    def zloop16(z, cz):
        pltpu.sync_copy(zbuf16, accdeg.at[pl.ds(r0 + z * ZR, ZR)])
        return cz
    lax.fori_loop(0, RPT // ZR, zloop16, 0)
    plsc.subcore_barrier()

    def deg_body(b, cd):
        pltpu.sync_copy(ones_v, accdeg.at[dst_t.at[b]], add=True)
        return cd
    lax.fori_loop(0, NBT, deg_body, 0)
    plsc.subcore_barrier()
    pltpu.sync_copy(accdeg.at[pl.ds(r0, RPT)],
                    outdeg_ref.at[c, pl.ds(r0, RPT)])
    plsc.subcore_barrier()


def _gcn_body(tab_ref, dinv_ref, srcb_ref, dstb_ref, out_ref,
              src_t, dst_t, rows0, rows1, ar0, ar1, br0, br1,
              zbuf, acc, sem0, sem1, sa0, sa1):
    c = lax.axis_index("c")
    s = lax.axis_index("s")
    base_blk = c * (NBLK // 2) + s * NBT
    r0 = s * RPT
    pltpu.sync_copy(srcb_ref.at[pl.ds(base_blk, NBT)], src_t)
    pltpu.sync_copy(dstb_ref.at[pl.ds(base_blk, NBT)], dst_t)
    for r in range(ZR):
        for j in range(D // 16):
            zbuf[r, pl.ds(j * 16, 16)] = jnp.zeros((16,), jnp.float32)

    def chunk_body(ci, carry):
        _zero_acc(zbuf, acc, r0)
        plsc.subcore_barrier()
        _pipeline_blocks(
            tab_ref.at[ci], dinv_ref, dinv_ref,
            lambda a, b: a * b,
            src_t, dst_t, (rows0, rows1), (ar0, ar1), (br0, br1),
            acc, (sem0, sem1), (sa0, sa1))
        plsc.subcore_barrier()
        pltpu.sync_copy(acc.at[pl.ds(r0, RPT)],
                        out_ref.at[c * H + ci, pl.ds(r0, RPT)])
        plsc.subcore_barrier()
        return carry
    lax.fori_loop(0, H, chunk_body, 0)


_SC_MESH = plsc.VectorSubcoreMesh(core_axis_name="c", subcore_axis_name="s")
_NO_TILING = pltpu.CompilerParams(use_tc_tiling_on_sc=False)

_SPMM_SCRATCH = [
    pltpu.VMEM((NBT, K), jnp.int32),
    pltpu.VMEM((NBT, K), jnp.int32),
    pltpu.VMEM((K, D), jnp.float32),
    pltpu.VMEM((K, D), jnp.float32),
    pltpu.VMEM((K, 16), jnp.float32),
    pltpu.VMEM((K, 16), jnp.float32),
    pltpu.VMEM((K, 16), jnp.float32),
    pltpu.VMEM((K, 16), jnp.float32),
]

_gat_spmm = pl.kernel(
    _gat_body,
    out_type=[
        jax.ShapeDtypeStruct((NSC * H, NPAD, D), jnp.float32),
        jax.ShapeDtypeStruct((NSC, NPAD, 16), jnp.float32),
    ],
    mesh=_SC_MESH,
    compiler_params=_NO_TILING,
    scratch_types=_SPMM_SCRATCH + [
        pltpu.VMEM((ZR, D), jnp.float32),
        pltpu.VMEM((ZR, 16), jnp.float32),
        pltpu.VMEM((K, 16), jnp.float32),
        pltpu.VMEM_SHARED((NPAD, D), jnp.float32),
        pltpu.VMEM_SHARED((NPAD, 16), jnp.float32),
        pltpu.SemaphoreType.DMA,
        pltpu.SemaphoreType.DMA,
        pltpu.SemaphoreType.DMA,
        pltpu.SemaphoreType.DMA,
    ],
)

_gcn_spmm = pl.kernel(
    _gcn_body,
    out_type=jax.ShapeDtypeStruct((NSC * H, NPAD, D), jnp.float32),
    mesh=_SC_MESH,
    compiler_params=_NO_TILING,
    scratch_types=_SPMM_SCRATCH + [
        pltpu.VMEM((ZR, D), jnp.float32),
        pltpu.VMEM_SHARED((NPAD, D), jnp.float32),
        pltpu.SemaphoreType.DMA,
        pltpu.SemaphoreType.DMA,
        pltpu.SemaphoreType.DMA,
        pltpu.SemaphoreType.DMA,
    ],
)


BR = 512


def _k1_body(x_ref, gl_ref, ws_ref, wd_ref, htab_ref, asrc_ref, adst_ref):
    i = pl.program_id(0)
    hb = jnp.dot(x_ref[...], gl_ref[...], preferred_element_type=jnp.float32, precision=jax.lax.Precision.HIGHEST)
    asb = jnp.dot(hb, ws_ref[...], preferred_element_type=jnp.float32, precision=jax.lax.Precision.HIGHEST)
    adb = jnp.dot(hb, wd_ref[...], preferred_element_type=jnp.float32, precision=jax.lax.Precision.HIGHEST)
    rows = i * BR + jax.lax.broadcasted_iota(jnp.int32, (BR, 1), 0)
    valid = rows < N
    asb = jnp.where(valid, asb, -1e9)
    adb = jnp.where(valid, adb, -1e9)
    onescol = jnp.where(valid, 1.0, 0.0).astype(jnp.float32)
    zcol = jnp.zeros((BR, 1), jnp.float32)
    for hd in range(H):
        htab_ref[hd, :, :F] = hb[:, hd * F:(hd + 1) * F]
        htab_ref[hd, :, F:F + 1] = onescol
        htab_ref[hd, :, F + 1:D] = zcol
        asrc_ref[hd] = jnp.broadcast_to(asb[:, hd:hd + 1], (BR, 16))
        adst_ref[hd] = jnp.broadcast_to(adb[:, hd:hd + 1], (BR, 16))


def _k1(xp, gat_lin, Ws, Wd):
    return pl.pallas_call(
        _k1_body,
        grid=(NPAD // BR,),
        in_specs=[
            pl.BlockSpec((BR, F), lambda i: (i, 0)),
            pl.BlockSpec((F, HF), lambda i: (0, 0)),
            pl.BlockSpec((HF, H), lambda i: (0, 0)),
            pl.BlockSpec((HF, H), lambda i: (0, 0)),
        ],
        out_specs=[
            pl.BlockSpec((H, BR, D), lambda i: (0, i, 0)),
            pl.BlockSpec((H, BR, 16), lambda i: (0, i, 0)),
            pl.BlockSpec((H, BR, 16), lambda i: (0, i, 0)),
        ],
        out_shape=[
            jax.ShapeDtypeStruct((H, NPAD, D), jnp.float32),
            jax.ShapeDtypeStruct((H, NPAD, 16), jnp.float32),
            jax.ShapeDtypeStruct((H, NPAD, 16), jnp.float32),
        ],
    )(xp, gat_lin, Ws, Wd)


def _k3_body(og_ref, odeg_ref, gb_ref, gw_ref, stab_ref, dinv_ref):
    pieces = []
    for hd in range(H):
        num = og_ref[hd, :, :F] + og_ref[H + hd, :, :F]
        den = og_ref[hd, :, F:F + 1] + og_ref[H + hd, :, F:F + 1] + 1e-16
        pieces.append(num / den)
    x1b = jnp.concatenate(pieces, axis=1) + gb_ref[...][None, :]
    x1b = jnp.maximum(x1b, 0.0)
    sb = jnp.dot(x1b, gw_ref[...], preferred_element_type=jnp.float32, precision=jax.lax.Precision.HIGHEST)
    zcol2 = jnp.zeros((BR, D - F), jnp.float32)
    for hd in range(H):
        stab_ref[hd, :, :F] = sb[:, hd * F:(hd + 1) * F]
        stab_ref[hd, :, F:D] = zcol2
    degb = odeg_ref[0, :, 0:1] + odeg_ref[1, :, 0:1]
    dinvb = jnp.where(degb > 0, jax.lax.rsqrt(degb), 0.0)
    dinv_ref[...] = jnp.broadcast_to(dinvb, (BR, 16))


def _k3(outg, outdeg, gat_bias, gcn_w):
    return pl.pallas_call(
        _k3_body,
        grid=(NPAD // BR,),
        in_specs=[
            pl.BlockSpec((NSC * H, BR, D), lambda i: (0, i, 0)),
            pl.BlockSpec((NSC, BR, 16), lambda i: (0, i, 0)),
            pl.BlockSpec((HF,), lambda i: (0,)),
            pl.BlockSpec((HF, HF), lambda i: (0, 0)),
        ],
        out_specs=[
            pl.BlockSpec((H, BR, D), lambda i: (0, i, 0)),
            pl.BlockSpec((BR, 16), lambda i: (i, 0)),
        ],
        out_shape=[
            jax.ShapeDtypeStruct((H, NPAD, D), jnp.float32),
            jax.ShapeDtypeStruct((NPAD, 16), jnp.float32),
        ],
    )(outg, outdeg, gat_bias, gcn_w)


def _dense_tail_body(p_ref, w1_ref, b1_ref, w2_ref, b2_ref,
                     inp_ref, g1_ref, g2_ref, at_ref, o_ref):
    t = jnp.dot(p_ref[...], w1_ref[...], preferred_element_type=jnp.float32, precision=jax.lax.Precision.HIGHEST)
    t = jnp.maximum(t + b1_ref[...][None, :], 0.0)
    t = jnp.dot(t, w2_ref[...], preferred_element_type=jnp.float32, precision=jax.lax.Precision.HIGHEST)
    t = t + b2_ref[...][None, :]
    at = at_ref[...]
    Dv = jax.lax.rsqrt(jnp.sum(at, axis=0, keepdims=True))
    adj = at * Dv.T * Dv
    y1 = jnp.dot(inp_ref[...], g1_ref[...], preferred_element_type=jnp.float32, precision=jax.lax.Precision.HIGHEST)
    y = jnp.dot(adj, y1, preferred_element_type=jnp.float32, precision=jax.lax.Precision.HIGHEST)
    y = jnp.where(y > 0, y, y * 0.2)
    y2 = jnp.dot(adj, jnp.dot(y, g2_ref[...], preferred_element_type=jnp.float32, precision=jax.lax.Precision.HIGHEST),
                 preferred_element_type=jnp.float32, precision=jax.lax.Precision.HIGHEST)
    o_ref[...] = jax.lax.dot_general(t, y2, (((1,), (1,)), ((), ())),
                                     preferred_element_type=jnp.float32, precision=jax.lax.Precision.HIGHEST)


def _dense_tail_body_y(p_ref, w1_ref, b1_ref, w2_ref, b2_ref, yt_ref, o_ref):
    t = jnp.dot(p_ref[...], w1_ref[...], preferred_element_type=jnp.float32, precision=jax.lax.Precision.HIGHEST)
    t = jnp.maximum(t + b1_ref[...][None, :], 0.0)
    t = jnp.dot(t, w2_ref[...], preferred_element_type=jnp.float32, precision=jax.lax.Precision.HIGHEST)
    t = t + b2_ref[...][None, :]
    o_ref[...] = jnp.dot(t, yt_ref[...], preferred_element_type=jnp.float32, precision=jax.lax.Precision.HIGHEST)


def _dense_tail_y(p, w1, b1, w2, b2, y):
    return pl.pallas_call(
        _dense_tail_body_y,
        out_shape=jax.ShapeDtypeStruct((G, C), jnp.float32),
    )(p, w1, b1, w2, b2, y.T)


def _gen_adj(A):
    Dd = jnp.power(A.sum(1), -0.5)
    Dm = jnp.diag(Dd)
    return jnp.matmul(jnp.matmul(A, Dm).T, Dm)


def _dense_tail(p, w1, b1, w2, b2, inp, g1, g2, A):
    return pl.pallas_call(
        _dense_tail_body,
        out_shape=jax.ShapeDtypeStruct((G, C), jnp.float32),
    )(p, w1, b1, w2, b2, inp, g1, g2, A.T)


def _unchunk(o):
    return o[:, :N, :F].transpose(1, 0, 2).reshape(N, HF)


def _replicate16(a):
    return jnp.broadcast_to(a[..., None], a.shape + (16,))


def kernel(x, edge_index, batch, inp, gat_lin, att_src, att_dst, gat_bias,
           gcn_w, gcn_b, fc1_w, fc1_b, fc2_w, fc2_b, gc1_w, gc2_w, A):
    n = x.shape[0]
    loop = jnp.arange(n, dtype=edge_index.dtype)
    src = jnp.concatenate([edge_index[0], loop])
    dst = jnp.concatenate([edge_index[1], loop])
    etot = src.shape[0]
    sentinel = N + jnp.arange(EPAD - etot, dtype=jnp.int32) % (NPAD - N)
    srcp = jnp.concatenate([src, sentinel]).reshape(NBLK, K)
    dstp = jnp.concatenate([dst, sentinel]).reshape(NBLK, K)
    xp = jnp.pad(x, ((0, NPAD - n), (0, 0)))
    blk = jnp.zeros((H, F, H), jnp.float32)
    Ws = blk.at[jnp.arange(H), :, jnp.arange(H)].set(att_src).reshape(HF, H)
    Wd = blk.at[jnp.arange(H), :, jnp.arange(H)].set(att_dst).reshape(HF, H)
    htab, asrcX, adstX = _k1(xp, gat_lin, Ws, Wd)
    outg, outdeg = _gat_spmm(htab, asrcX, adstX, srcp, dstp)
    stab, dinvX = _k3(outg, outdeg, gat_bias, gcn_w)
    out = _gcn_spmm(stab, dinvX, srcp, dstp)
    x2 = _unchunk(out[:H] + out[H:]) + gcn_b
    x2 = jax.nn.relu(x2)
    gm = jax.ops.segment_max(x2, batch, num_segments=G)
    gm = jnp.where(jnp.isfinite(gm), gm, 0.0)
    cnt = jax.ops.segment_sum(jnp.ones((n,), dtype=jnp.float32), batch, num_segments=G)
    ga = jax.ops.segment_sum(x2, batch, num_segments=G) / jnp.maximum(cnt, 1.0)[:, None]
    p = jnp.concatenate([gm, ga], axis=1)
    adj = jax.lax.stop_gradient(_gen_adj(A))
    y = jnp.matmul(adj, jnp.matmul(inp, gc1_w))
    y = jax.nn.leaky_relu(y, negative_slope=0.2)
    y = jnp.matmul(adj, jnp.matmul(y, gc2_w))
    return _dense_tail_y(p, fc1_w, fc1_b, fc2_w, fc2_b, y)
